# Initial kernel scaffold; baseline (speedup 1.0000x reference)
import jax
import jax.numpy as jnp
from jax.experimental import pallas as pl


def kernel(sequence_ids, item_embeddings, W_llm, b_llm, W_gate, b_gate, W_proj, b_proj):
    raise NotImplementedError("write your pallas kernel here")


# speedup vs baseline: 2.6743x; 2.6743x over previous
import functools

import jax
import jax.numpy as jnp
from jax import lax
from jax.experimental import pallas as pl
from jax.experimental.pallas import tpu as pltpu
from jax.experimental.pallas import tpu_sc as plsc

B, L, VOCAB, D, K = 1024, 50, 100000, 128, 20

NC, NS = 2, 16
NW = NC * NS
LP = 56
KP = 24

ROWS_PER_W = B // NW
CHUNK = 8

RB = 256
VT = 2048
NVT = 49
VOCABP = NVT * VT
SEG = 128
NSEG = VOCABP // SEG
NCAND = KP * SEG

GROWS = B * KP // NW
GCHUNK = 384


def _make_sc_pool(n_ids: int):
    n_chunk_ids = CHUNK * n_ids
    mesh = plsc.VectorSubcoreMesh(core_axis_name="c", subcore_axis_name="s")

    @functools.partial(
        pl.kernel,
        mesh=mesh,
        out_type=jax.ShapeDtypeStruct((B, D), jnp.float32),
        scratch_types=[
            pltpu.VMEM((n_chunk_ids,), jnp.int32),
            pltpu.VMEM((n_chunk_ids, D), jnp.float32),
            pltpu.VMEM((ROWS_PER_W, D), jnp.float32),
            pltpu.SemaphoreType.DMA,
        ],
    )
    def pool(idx_hbm, table_hbm, out_hbm, idx_v, rows_v, acc_v, sem):
        wid = lax.axis_index("s") * NC + lax.axis_index("c")
        row_base = wid * ROWS_PER_W
        for c in range(ROWS_PER_W // CHUNK):
            pltpu.sync_copy(
                idx_hbm.at[pl.ds((row_base + c * CHUNK) * n_ids, n_chunk_ids)],
                idx_v,
            )
            pltpu.async_copy(table_hbm.at[idx_v], rows_v, sem).wait()

            def row_body(r, _):
                for d in range(D // 16):
                    def add_body(i, a):
                        return a + rows_v[r * n_ids + i, pl.ds(d * 16, 16)]
                    acc = lax.fori_loop(
                        0, n_ids, add_body, jnp.zeros((16,), jnp.float32)
                    )
                    acc_v[c * CHUNK + r, pl.ds(d * 16, 16)] = acc
                return 0

            lax.fori_loop(0, CHUNK, row_body, 0)
        pltpu.sync_copy(acc_v, out_hbm.at[pl.ds(row_base, ROWS_PER_W)])

    return pool


_sc_pool_seq = _make_sc_pool(LP)
_sc_pool_ret = _make_sc_pool(KP)


def _sc_gather_segments():
    mesh = plsc.VectorSubcoreMesh(core_axis_name="c", subcore_axis_name="s")

    @functools.partial(
        pl.kernel,
        mesh=mesh,
        out_type=jax.ShapeDtypeStruct((B * KP, SEG), jnp.float32),
        scratch_types=[
            pltpu.VMEM((GCHUNK,), jnp.int32),
            pltpu.VMEM((GCHUNK, SEG), jnp.float32),
            pltpu.SemaphoreType.DMA,
        ],
    )
    def gather(idx_hbm, table_hbm, out_hbm, idx_v, rows_v, sem):
        wid = lax.axis_index("s") * NC + lax.axis_index("c")
        for c in range(GROWS // GCHUNK):
            base = wid * GROWS + c * GCHUNK
            pltpu.sync_copy(idx_hbm.at[pl.ds(base, GCHUNK)], idx_v)
            pltpu.async_copy(table_hbm.at[idx_v], rows_v, sem).wait()
            pltpu.sync_copy(rows_v, out_hbm.at[pl.ds(base, GCHUNK)])

    return gather


_sc_gather = _sc_gather_segments()


def _user_rep_body(pooled_ref, w_ref, b_ref, out_ref):
    out_ref[...] = jnp.tanh(
        jnp.dot(pooled_ref[...], w_ref[...], preferred_element_type=jnp.float32)
        + b_ref[...]
    )


def _user_rep(pooled, W_llm, b_llm):
    return pl.pallas_call(
        _user_rep_body,
        out_shape=jax.ShapeDtypeStruct((B, D), jnp.float32),
    )(pooled, W_llm, b_llm.reshape(1, D))


def _scores_body(ur_ref, e_ref, scores_ref, segmax_ref):
    j = pl.program_id(1)
    s = lax.dot_general(
        ur_ref[...], e_ref[...],
        (((1,), (1,)), ((), ())),
        preferred_element_type=jnp.float32,
    )
    col = j * VT + lax.broadcasted_iota(jnp.int32, (RB, VT), 1)
    s = jnp.where(col < VOCAB, s, -jnp.inf)
    scores_ref[...] = s
    maxes = [
        jnp.max(s[:, t * SEG:(t + 1) * SEG], axis=1, keepdims=True)
        for t in range(VT // SEG)
    ]
    segmax_ref[0] = jnp.concatenate(maxes, axis=1)


def _scores_segmax(user_rep, item_embeddings):
    return pl.pallas_call(
        _scores_body,
        grid=(B // RB, NVT),
        in_specs=[
            pl.BlockSpec((RB, D), lambda i, j: (i, 0)),
            pl.BlockSpec((VT, D), lambda i, j: (j, 0)),
        ],
        out_specs=[
            pl.BlockSpec((RB, VT), lambda i, j: (i, j)),
            pl.BlockSpec((1, RB, VT // SEG), lambda i, j: (j, i, 0)),
        ],
        out_shape=[
            jax.ShapeDtypeStruct((B, VOCABP), jnp.float32),
            jax.ShapeDtypeStruct((NVT, B, VT // SEG), jnp.float32),
        ],
    )(user_rep, item_embeddings)


def _top_segs_body(segmax_ref, out_ref):
    vals = segmax_ref[...]
    segid = lax.broadcasted_iota(jnp.int32, (RB, NSEG), 1)
    picked = []
    for _ in range(K):
        m = jnp.max(vals, axis=1, keepdims=True)
        sel = jnp.where(vals == m, segid, jnp.int32(2 ** 30))
        bid = jnp.min(sel, axis=1, keepdims=True)
        picked.append(bid)
        vals = jnp.where(segid == bid, -jnp.inf, vals)
    out_ref[...] = jnp.concatenate(picked, axis=1)


def _top_segments(segmax):
    return pl.pallas_call(
        _top_segs_body,
        grid=(B // RB,),
        in_specs=[pl.BlockSpec((RB, NSEG), lambda i: (i, 0))],
        out_specs=pl.BlockSpec((RB, K), lambda i: (i, 0)),
        out_shape=jax.ShapeDtypeStruct((B, K), jnp.int32),
    )(segmax)


def _top_cands_body(cands_ref, cid_ref, out_ref):
    vals = cands_ref[...]
    cid = cid_ref[...]
    picked = []
    for _ in range(K):
        m = jnp.max(vals, axis=1, keepdims=True)
        sel = jnp.where(vals == m, cid, jnp.int32(2 ** 30))
        bid = jnp.min(sel, axis=1, keepdims=True)
        picked.append(bid)
        vals = jnp.where(cid == bid, -jnp.inf, vals)
    out_ref[...] = jnp.concatenate(picked, axis=1)


def _top_candidates(cands, cid):
    return pl.pallas_call(
        _top_cands_body,
        grid=(B // RB,),
        in_specs=[
            pl.BlockSpec((RB, NCAND), lambda i: (i, 0)),
            pl.BlockSpec((RB, NCAND), lambda i: (i, 0)),
        ],
        out_specs=pl.BlockSpec((RB, K), lambda i: (i, 0)),
        out_shape=jax.ShapeDtypeStruct((B, K), jnp.int32),
    )(cands, cid)


def _fuse_proj_body(ur_ref, re_ref, wg_ref, bg_ref, wp_ref, bp_ref, out_ref):
    ur = ur_ref[...]
    re = re_ref[...]
    g = jax.nn.sigmoid(
        jnp.dot(ur, wg_ref[:D, :], preferred_element_type=jnp.float32)
        + jnp.dot(re, wg_ref[D:, :], preferred_element_type=jnp.float32)
        + bg_ref[...]
    )
    fused = g * ur + (1.0 - g) * re
    out_ref[...] = (
        jnp.dot(fused, wp_ref[...], preferred_element_type=jnp.float32)
        + bp_ref[...]
    )


def _fuse_proj(user_rep, retrieved, W_gate, b_gate, W_proj, b_proj):
    return pl.pallas_call(
        _fuse_proj_body,
        grid=(B // RB, NVT),
        in_specs=[
            pl.BlockSpec((RB, D), lambda i, j: (i, 0)),
            pl.BlockSpec((RB, D), lambda i, j: (i, 0)),
            pl.BlockSpec((2 * D, D), lambda i, j: (0, 0)),
            pl.BlockSpec((1, D), lambda i, j: (0, 0)),
            pl.BlockSpec((D, VT), lambda i, j: (0, j)),
            pl.BlockSpec((1, VT), lambda i, j: (0, j)),
        ],
        out_specs=pl.BlockSpec((RB, VT), lambda i, j: (i, j)),
        out_shape=jax.ShapeDtypeStruct((B, VOCAB), jnp.float32),
    )(user_rep, retrieved, W_gate, b_gate.reshape(1, D), W_proj,
      b_proj.reshape(1, VOCAB))


def kernel(sequence_ids, item_embeddings, W_llm, b_llm, W_gate, b_gate,
           W_proj, b_proj):
    mask = sequence_ids == 0
    cnt0 = jnp.sum(mask, axis=1).astype(jnp.float32)
    denom = jnp.maximum(jnp.float32(L) - cnt0, 1.0)
    seq_ids = jnp.where(mask, 0, sequence_ids - 1).astype(jnp.int32)
    seq_ids = jnp.pad(seq_ids, ((0, 0), (0, LP - L)))
    e0 = item_embeddings[0]

    pooled_sum = _sc_pool_seq(seq_ids.reshape(-1), item_embeddings)
    pooled = (pooled_sum - (cnt0 + (LP - L))[:, None] * e0[None, :]) \
        / denom[:, None]

    user_rep = _user_rep(pooled, W_llm, b_llm)

    scores, segmax3 = _scores_segmax(user_rep, item_embeddings)
    segmax = jnp.transpose(segmax3, (1, 0, 2)).reshape(B, NSEG)
    segids = _top_segments(segmax)

    segids_p = jnp.pad(segids, ((0, 0), (0, KP - K)))
    seg_rows = (jnp.arange(B, dtype=jnp.int32)[:, None] * NSEG + segids_p)
    cands = _sc_gather(seg_rows.reshape(-1),
                       scores.reshape(B * NSEG, SEG))
    cands = cands.reshape(B, NCAND)
    cid = (segids_p[:, :, None] * SEG
           + jnp.arange(SEG, dtype=jnp.int32)[None, None, :]).reshape(B, NCAND)
    topk_idx = _top_candidates(cands, cid)

    topk_pad = jnp.pad(topk_idx, ((0, 0), (0, KP - K)))
    ret_sum = _sc_pool_ret(topk_pad.reshape(-1), item_embeddings)
    retrieved = (ret_sum - jnp.float32(KP - K) * e0[None, :]) / jnp.float32(K)

    return _fuse_proj(user_rep, retrieved, W_gate, b_gate, W_proj, b_proj)


# speedup vs baseline: 2.7240x; 1.0186x over previous
import functools

import jax
import jax.numpy as jnp
from jax import lax
from jax.experimental import pallas as pl
from jax.experimental.pallas import tpu as pltpu
from jax.experimental.pallas import tpu_sc as plsc

B, L, VOCAB, D, K = 1024, 50, 100000, 128, 20

NC, NS = 2, 16
NW = NC * NS
LP = 56
KP = 24

ROWS_PER_W = B // NW
CHUNK = 8

RB = 256
VT = 2048
NVT = 49
VOCABP = NVT * VT
SEG = 128
NSEG = VOCABP // SEG
NCAND = K * SEG

GROWS = B * K // NW
GCHUNK = 320


def _make_sc_pool(n_ids: int):
    n_chunk_ids = CHUNK * n_ids
    mesh = plsc.VectorSubcoreMesh(core_axis_name="c", subcore_axis_name="s")

    n_chunks = ROWS_PER_W // CHUNK

    @functools.partial(
        pl.kernel,
        mesh=mesh,
        out_type=jax.ShapeDtypeStruct((B, D), jnp.float32),
        scratch_types=[
            pltpu.VMEM((n_chunk_ids,), jnp.int32),
            pltpu.VMEM((n_chunk_ids, D), jnp.float32),
            pltpu.VMEM((n_chunk_ids,), jnp.int32),
            pltpu.VMEM((n_chunk_ids, D), jnp.float32),
            pltpu.VMEM((ROWS_PER_W, D), jnp.float32),
            pltpu.SemaphoreType.DMA,
            pltpu.SemaphoreType.DMA,
        ],
    )
    def pool(idx_hbm, table_hbm, out_hbm, idx_v0, rows_v0, idx_v1, rows_v1,
             acc_v, sem0, sem1):
        wid = lax.axis_index("s") * NC + lax.axis_index("c")
        row_base = wid * ROWS_PER_W
        idx_vs, rows_vs, sems = [idx_v0, idx_v1], [rows_v0, rows_v1], \
            [sem0, sem1]

        def start(c):
            pltpu.sync_copy(
                idx_hbm.at[pl.ds((row_base + c * CHUNK) * n_ids, n_chunk_ids)],
                idx_vs[c % 2],
            )
            return pltpu.async_copy(
                table_hbm.at[idx_vs[c % 2]], rows_vs[c % 2], sems[c % 2]
            )

        cp = start(0)
        for c in range(n_chunks):
            nxt = start(c + 1) if c + 1 < n_chunks else None
            cp.wait()
            rows_v = rows_vs[c % 2]

            def row_body(r, _):
                base = r * n_ids
                for d in range(D // 16):
                    parts = []
                    for p in range(4):
                        a = rows_v[base + p, pl.ds(d * 16, 16)]
                        for i in range(p + 4, n_ids, 4):
                            a = a + rows_v[base + i, pl.ds(d * 16, 16)]
                        parts.append(a)
                    acc_v[c * CHUNK + r, pl.ds(d * 16, 16)] = (
                        (parts[0] + parts[1]) + (parts[2] + parts[3])
                    )
                return 0

            lax.fori_loop(0, CHUNK, row_body, 0)
            cp = nxt
        pltpu.sync_copy(acc_v, out_hbm.at[pl.ds(row_base, ROWS_PER_W)])

    return pool


_sc_pool_seq = _make_sc_pool(LP)
_sc_pool_ret = _make_sc_pool(KP)


def _sc_gather_segments():
    mesh = plsc.VectorSubcoreMesh(core_axis_name="c", subcore_axis_name="s")

    @functools.partial(
        pl.kernel,
        mesh=mesh,
        out_type=jax.ShapeDtypeStruct((B * K, SEG), jnp.float32),
        scratch_types=[
            pltpu.VMEM((GCHUNK,), jnp.int32),
            pltpu.VMEM((GCHUNK, SEG), jnp.float32),
            pltpu.VMEM((GCHUNK,), jnp.int32),
            pltpu.VMEM((GCHUNK, SEG), jnp.float32),
            pltpu.SemaphoreType.DMA,
            pltpu.SemaphoreType.DMA,
        ],
    )
    def gather(idx_hbm, table_hbm, out_hbm, idx_v0, rows_v0, idx_v1, rows_v1,
               sem0, sem1):
        wid = lax.axis_index("s") * NC + lax.axis_index("c")
        idx_vs, rows_vs, sems = [idx_v0, idx_v1], [rows_v0, rows_v1], \
            [sem0, sem1]
        n_chunks = GROWS // GCHUNK

        def start(c):
            pltpu.sync_copy(
                idx_hbm.at[pl.ds(wid * GROWS + c * GCHUNK, GCHUNK)],
                idx_vs[c % 2],
            )
            return pltpu.async_copy(
                table_hbm.at[idx_vs[c % 2]], rows_vs[c % 2], sems[c % 2]
            )

        cp = start(0)
        for c in range(n_chunks):
            nxt = start(c + 1) if c + 1 < n_chunks else None
            cp.wait()
            pltpu.sync_copy(
                rows_vs[c % 2],
                out_hbm.at[pl.ds(wid * GROWS + c * GCHUNK, GCHUNK)],
            )
            cp = nxt

    return gather


_sc_gather = _sc_gather_segments()


def _user_rep_body(pooled_ref, w_ref, b_ref, out_ref):
    out_ref[...] = jnp.tanh(
        jnp.dot(pooled_ref[...], w_ref[...], preferred_element_type=jnp.float32)
        + b_ref[...]
    )


def _user_rep(pooled, W_llm, b_llm):
    return pl.pallas_call(
        _user_rep_body,
        out_shape=jax.ShapeDtypeStruct((B, D), jnp.float32),
    )(pooled, W_llm, b_llm.reshape(1, D))


def _scores_body(ur_ref, e_ref, scores_ref, segmax_ref):
    j = pl.program_id(1)
    s = lax.dot_general(
        ur_ref[...], e_ref[...],
        (((1,), (1,)), ((), ())),
        preferred_element_type=jnp.float32,
    )
    col = j * VT + lax.broadcasted_iota(jnp.int32, (RB, VT), 1)
    s = jnp.where(col < VOCAB, s, -jnp.inf)
    scores_ref[...] = s
    maxes = [
        jnp.max(s[:, t * SEG:(t + 1) * SEG], axis=1, keepdims=True)
        for t in range(VT // SEG)
    ]
    segmax_ref[0] = jnp.concatenate(maxes, axis=1)


def _scores_segmax(user_rep, item_embeddings):
    return pl.pallas_call(
        _scores_body,
        grid=(B // RB, NVT),
        in_specs=[
            pl.BlockSpec((RB, D), lambda i, j: (i, 0)),
            pl.BlockSpec((VT, D), lambda i, j: (j, 0)),
        ],
        out_specs=[
            pl.BlockSpec((RB, VT), lambda i, j: (i, j)),
            pl.BlockSpec((1, RB, VT // SEG), lambda i, j: (j, i, 0)),
        ],
        out_shape=[
            jax.ShapeDtypeStruct((B, VOCABP), jnp.float32),
            jax.ShapeDtypeStruct((NVT, B, VT // SEG), jnp.float32),
        ],
    )(user_rep, item_embeddings)


def _top_segs_body(segmax_ref, out_ref):
    vals = segmax_ref[...]
    segid = lax.broadcasted_iota(jnp.int32, (RB, NSEG), 1)
    picked = []
    for _ in range(K):
        m = jnp.max(vals, axis=1, keepdims=True)
        sel = jnp.where(vals == m, segid, jnp.int32(2 ** 30))
        bid = jnp.min(sel, axis=1, keepdims=True)
        picked.append(bid)
        vals = jnp.where(segid == bid, -jnp.inf, vals)
    out_ref[...] = jnp.concatenate(picked, axis=1)


def _top_segments(segmax):
    return pl.pallas_call(
        _top_segs_body,
        grid=(B // RB,),
        in_specs=[pl.BlockSpec((RB, NSEG), lambda i: (i, 0))],
        out_specs=pl.BlockSpec((RB, K), lambda i: (i, 0)),
        out_shape=jax.ShapeDtypeStruct((B, K), jnp.int32),
    )(segmax)


def _top_cands_body(cands_ref, cid_ref, out_ref):
    vals = cands_ref[...]
    cid = cid_ref[...]
    picked = []
    for _ in range(K):
        m = jnp.max(vals, axis=1, keepdims=True)
        sel = jnp.where(vals == m, cid, jnp.int32(2 ** 30))
        bid = jnp.min(sel, axis=1, keepdims=True)
        picked.append(bid)
        vals = jnp.where(cid == bid, -jnp.inf, vals)
    out_ref[...] = jnp.concatenate(picked, axis=1)


def _top_candidates(cands, cid):
    return pl.pallas_call(
        _top_cands_body,
        grid=(B // RB,),
        in_specs=[
            pl.BlockSpec((RB, NCAND), lambda i: (i, 0)),
            pl.BlockSpec((RB, NCAND), lambda i: (i, 0)),
        ],
        out_specs=pl.BlockSpec((RB, K), lambda i: (i, 0)),
        out_shape=jax.ShapeDtypeStruct((B, K), jnp.int32),
    )(cands, cid)


def _fuse_proj_body(ur_ref, re_ref, wg_ref, bg_ref, wp_ref, bp_ref, out_ref):
    ur = ur_ref[...]
    re = re_ref[...]
    g = jax.nn.sigmoid(
        jnp.dot(ur, wg_ref[:D, :], preferred_element_type=jnp.float32)
        + jnp.dot(re, wg_ref[D:, :], preferred_element_type=jnp.float32)
        + bg_ref[...]
    )
    fused = g * ur + (1.0 - g) * re
    out_ref[...] = (
        jnp.dot(fused, wp_ref[...], preferred_element_type=jnp.float32)
        + bp_ref[...]
    )


def _fuse_proj(user_rep, retrieved, W_gate, b_gate, W_proj, b_proj):
    return pl.pallas_call(
        _fuse_proj_body,
        grid=(B // RB, NVT),
        in_specs=[
            pl.BlockSpec((RB, D), lambda i, j: (i, 0)),
            pl.BlockSpec((RB, D), lambda i, j: (i, 0)),
            pl.BlockSpec((2 * D, D), lambda i, j: (0, 0)),
            pl.BlockSpec((1, D), lambda i, j: (0, 0)),
            pl.BlockSpec((D, VT), lambda i, j: (0, j)),
            pl.BlockSpec((1, VT), lambda i, j: (0, j)),
        ],
        out_specs=pl.BlockSpec((RB, VT), lambda i, j: (i, j)),
        out_shape=jax.ShapeDtypeStruct((B, VOCAB), jnp.float32),
    )(user_rep, retrieved, W_gate, b_gate.reshape(1, D), W_proj,
      b_proj.reshape(1, VOCAB))


def kernel(sequence_ids, item_embeddings, W_llm, b_llm, W_gate, b_gate,
           W_proj, b_proj):
    mask = sequence_ids == 0
    cnt0 = jnp.sum(mask, axis=1).astype(jnp.float32)
    denom = jnp.maximum(jnp.float32(L) - cnt0, 1.0)
    seq_ids = jnp.where(mask, 0, sequence_ids - 1).astype(jnp.int32)
    seq_ids = jnp.pad(seq_ids, ((0, 0), (0, LP - L)))
    e0 = item_embeddings[0]

    pooled_sum = _sc_pool_seq(seq_ids.reshape(-1), item_embeddings)
    pooled = (pooled_sum - (cnt0 + (LP - L))[:, None] * e0[None, :]) \
        / denom[:, None]

    user_rep = _user_rep(pooled, W_llm, b_llm)

    scores, segmax3 = _scores_segmax(user_rep, item_embeddings)
    segmax = jnp.transpose(segmax3, (1, 0, 2)).reshape(B, NSEG)
    segids = _top_segments(segmax)

    seg_rows = (jnp.arange(B, dtype=jnp.int32)[:, None] * NSEG + segids)
    cands = _sc_gather(seg_rows.reshape(-1),
                       scores.reshape(B * NSEG, SEG))
    cands = cands.reshape(B, NCAND)
    cid = (segids[:, :, None] * SEG
           + jnp.arange(SEG, dtype=jnp.int32)[None, None, :]).reshape(B, NCAND)
    topk_idx = _top_candidates(cands, cid)

    topk_pad = jnp.pad(topk_idx, ((0, 0), (0, KP - K)))
    ret_sum = _sc_pool_ret(topk_pad.reshape(-1), item_embeddings)
    retrieved = (ret_sum - jnp.float32(KP - K) * e0[None, :]) / jnp.float32(K)

    return _fuse_proj(user_rep, retrieved, W_gate, b_gate, W_proj, b_proj)
